# baseline (speedup 1.0000x reference)
import functools

import jax
import jax.numpy as jnp
from jax import lax
from jax.experimental import pallas as pl
from jax.experimental.pallas import tpu as pltpu
from jax.experimental.pallas import tpu_sc as plsc

_B, _N, _S, _K, _C = 8, 4096, 512, 32, 128


_QB = 256
_XYZ_PAD = 16


def _fps_body(pts_ref, out_ref, q_ref, iota_ref, col_ref):
    xs = pts_ref[0:_B, :]
    ys = pts_ref[_B:2 * _B, :]
    zs = pts_ref[2 * _B:3 * _B, :]
    iota_ref[...] = lax.broadcasted_iota(
        jnp.int32, (_B, _N), 1).astype(jnp.float32)
    col_ref[...] = lax.broadcasted_iota(
        jnp.int32, (_B, _S), 1).astype(jnp.float32)
    iota = iota_ref[...]
    col = col_ref[...]
    nf = jnp.float32(_N)

    def body(t, carry):
        dists, far, cx, cy, cz, outv, lcx, lcy, lcz = carry
        hitf = (col == t.astype(jnp.float32)).astype(jnp.float32)
        outv = outv + hitf * jnp.broadcast_to(far, (_B, _S))
        lcx = lcx + hitf * jnp.broadcast_to(cx, (_B, _S))
        lcy = lcy + hitf * jnp.broadcast_to(cy, (_B, _S))
        lcz = lcz + hitf * jnp.broadcast_to(cz, (_B, _S))
        dx = xs - jnp.broadcast_to(cx, (_B, _N))
        dy = ys - jnp.broadcast_to(cy, (_B, _N))
        dz = zs - jnp.broadcast_to(cz, (_B, _N))
        d = dx * dx + dy * dy + dz * dz
        dists = jnp.minimum(dists, d)
        dv, iv, xv, yv, zv = dists, iota, xs, ys, zs
        w = _N
        while w > 128:
            h = w // 2
            dl, dr = dv[:, :h], dv[:, h:]
            il, ir = iv[:, :h], iv[:, h:]
            take_r = (dr > dl) | ((dr == dl) & (ir < il))
            dv = jnp.where(take_r, dr, dl)
            iv = jnp.where(take_r, ir, il)
            xv = jnp.where(take_r, xv[:, h:], xv[:, :h])
            yv = jnp.where(take_r, yv[:, h:], yv[:, :h])
            zv = jnp.where(take_r, zv[:, h:], zv[:, :h])
            w = h
        m = jnp.max(dv, axis=1, keepdims=True)
        candm = jnp.where(dv == jnp.broadcast_to(m, (_B, 128)), iv, nf)
        far2 = jnp.min(candm, axis=1, keepdims=True)
        pos = (iv == jnp.broadcast_to(far2, (_B, 128))).astype(jnp.float32)
        cx2 = jnp.sum(pos * xv, axis=1, keepdims=True)
        cy2 = jnp.sum(pos * yv, axis=1, keepdims=True)
        cz2 = jnp.sum(pos * zv, axis=1, keepdims=True)
        return dists, far2, cx2, cy2, cz2, outv, lcx, lcy, lcz

    dists0 = jnp.full((_B, _N), 1e10, jnp.float32)
    far0 = jnp.zeros((_B, 1), jnp.float32)
    cx0 = xs[:, 0:1]
    cy0 = ys[:, 0:1]
    cz0 = zs[:, 0:1]
    z = jnp.zeros((_B, _S), jnp.float32)
    out = lax.fori_loop(
        0, _S, body, (dists0, far0, cx0, cy0, cz0, z, z, z, z))
    _, _, _, _, _, outv, lcx, lcy, lcz = out
    out_ref[...] = outv.astype(jnp.int32)
    q_ref[0:_B, :] = lcx
    q_ref[_B:2 * _B, :] = lcy
    q_ref[2 * _B:3 * _B, :] = lcz


def _fps(pts):
    return pl.pallas_call(
        _fps_body,
        out_shape=(
            jax.ShapeDtypeStruct((_B, _S), jnp.int32),
            jax.ShapeDtypeStruct((3 * _B, _S), jnp.float32),
        ),
        scratch_shapes=[
            pltpu.VMEM((_B, _N), jnp.float32),
            pltpu.VMEM((_B, _S), jnp.float32),
        ],
    )(pts)


def _knn_body(q_ref, p_ref, out_ref, dist_ref, iota_ref, kcol_ref):
    px = p_ref[0, 0]
    py = p_ref[1, 0]
    pz = p_ref[2, 0]
    qx = q_ref[0, 0]
    qy = q_ref[1, 0]
    qz = q_ref[2, 0]
    def rb(v):
        return v.astype(jnp.bfloat16).astype(jnp.float32)

    dot = (rb(qx) * rb(px) + rb(qy) * rb(py)) + rb(qz) * rb(pz)
    q2 = (qx * qx + qy * qy) + qz * qz
    p2 = (px * px + py * py) + pz * pz
    dist_ref[...] = (-2.0 * dot + q2) + p2

    iota_ref[...] = lax.broadcasted_iota(
        jnp.int32, (_QB, _N), 1).astype(jnp.float32)
    kcol_ref[...] = lax.broadcasted_iota(jnp.int32, (_QB, _K), 1)
    iota = iota_ref[...]
    kcol = kcol_ref[...]
    big = jnp.float32(1e30)
    nf = jnp.float32(_N)

    def body(k, outv):
        d = dist_ref[...]
        m = jnp.min(d, axis=1, keepdims=True)
        cand = jnp.where(d == jnp.broadcast_to(m, (_QB, _N)), iota, nf)
        idx = jnp.min(cand, axis=1, keepdims=True)
        hiti = (kcol == k).astype(jnp.int32)
        outv = outv + hiti * jnp.broadcast_to(
            idx.astype(jnp.int32), (_QB, _K))
        picked = jnp.broadcast_to(idx, (_QB, _N)) == iota
        dist_ref[...] = jnp.where(picked, big, d)
        return outv

    outv = lax.fori_loop(0, _K, body, jnp.zeros((_QB, _K), jnp.int32))
    out_ref[...] = outv.reshape(1, _QB, _K)


def _knn(q, pts4, nb=_B):
    return pl.pallas_call(
        _knn_body,
        grid=(nb, _S // _QB),
        in_specs=[
            pl.BlockSpec((3, 1, _QB, 1), lambda b, j: (0, b, j, 0)),
            pl.BlockSpec((3, 1, 1, _N), lambda b, j: (0, b, 0, 0)),
        ],
        out_specs=pl.BlockSpec((1, _QB, _K), lambda b, j: (b, j, 0)),
        out_shape=jax.ShapeDtypeStruct((nb, _S, _K), jnp.int32),
        scratch_shapes=[
            pltpu.VMEM((_QB, _N), jnp.float32),
            pltpu.VMEM((_QB, _N), jnp.float32),
            pltpu.VMEM((_QB, _K), jnp.int32),
        ],
    )(q, pts4)


@functools.lru_cache(maxsize=None)
def _make_sc_gather(n_idx, n_tables):
    info = plsc.get_sparse_core_info()
    nc, ns = info.num_cores, info.num_subcores
    nw = nc * ns
    per_w = n_idx // nw
    ch = min(per_w, 128)
    chunks = per_w // ch
    mesh = plsc.VectorSubcoreMesh(core_axis_name="c", subcore_axis_name="s")

    @functools.partial(
        pl.kernel,
        mesh=mesh,
        out_type=tuple(
            jax.ShapeDtypeStruct((n_idx, _C), jnp.float32)
            for _ in range(n_tables)
        ),
        scratch_types=(
            [pltpu.VMEM((ch,), jnp.int32)]
            + [pltpu.VMEM((ch, _C), jnp.float32) for _ in range(n_tables)]
            + [pltpu.SemaphoreType.DMA for _ in range(n_tables)]
        ),
    )
    def k(idx_hbm, *rest):
        tables = rest[:n_tables]
        outs = rest[n_tables:2 * n_tables]
        idx_v = rest[2 * n_tables]
        rows = rest[2 * n_tables + 1:2 * n_tables + 1 + n_tables]
        sems = rest[2 * n_tables + 1 + n_tables:]
        wid = lax.axis_index("s") * nc + lax.axis_index("c")
        base = wid * per_w

        def body(c, carry):
            off = base + c * ch
            pltpu.sync_copy(idx_hbm.at[pl.ds(off, ch)], idx_v)
            cps = [pltpu.async_copy(t.at[idx_v], r, s)
                   for t, r, s in zip(tables, rows, sems)]
            for cp in cps:
                cp.wait()
            for r, o in zip(rows, outs):
                pltpu.sync_copy(r, o.at[pl.ds(off, ch)])
            return carry

        lax.fori_loop(0, chunks, body, 0)

    return k


def kernel(xyz, x):
    pts = jnp.transpose(xyz, (2, 0, 1)).reshape(3 * _B, _N)
    pts4 = pts.reshape(3, _B, 1, _N)
    x2d = x.reshape(_B * _N, _C)
    xyz128 = jnp.pad(xyz, ((0, 0), (0, 0), (0, _C - 3)))
    xyz128 = xyz128.reshape(_B * _N, _C)
    row_off = jnp.arange(_B, dtype=jnp.int32) * _N

    fps_idx, qpts = _fps(pts)
    lc_xyz = qpts.reshape(3, _B, _S).transpose(1, 2, 0)

    gidx_lc = (fps_idx + row_off[:, None]).reshape(-1)
    (lc_x_flat,) = _make_sc_gather(_B * _S, 1)(gidx_lc, x2d)
    lc_x = lc_x_flat.reshape(_B, _S, _C)

    q = qpts.reshape(3, _B, _S, 1)
    hb = _B // 2
    gx_halves, gxyz_halves = [], []
    for h in range(2):
        bs = slice(h * hb, (h + 1) * hb)
        knn_idx_h = _knn(q[:, bs], pts4[:, bs], hb)
        gidx_h = (knn_idx_h + row_off[bs][:, None, None]).reshape(-1)
        gx, gxyz = _make_sc_gather(hb * _S * _K, 2)(gidx_h, x2d, xyz128)
        gx_halves.append(gx)
        gxyz_halves.append(gxyz)
    knn_x = jnp.concatenate(gx_halves).reshape(_B, _S, _K, _C)
    knn_xyz = jnp.concatenate(gxyz_halves)[:, :3].reshape(_B, _S, _K, 3)

    return (lc_xyz, lc_x, knn_xyz, knn_x)


# speedup vs baseline: 1.1132x; 1.1132x over previous
import functools

import jax
import jax.numpy as jnp
from jax import lax
from jax.experimental import pallas as pl
from jax.experimental.pallas import tpu as pltpu
from jax.experimental.pallas import tpu_sc as plsc

_B, _N, _S, _K, _C = 8, 4096, 512, 32, 128


_QB = 256
_XYZ_PAD = 16


def _fps_body(pts_ref, out_ref, q_ref, iota_ref, col_ref):
    xs = pts_ref[0:_B, :]
    ys = pts_ref[_B:2 * _B, :]
    zs = pts_ref[2 * _B:3 * _B, :]
    iota_ref[...] = lax.broadcasted_iota(
        jnp.int32, (_B, _N), 1).astype(jnp.float32)
    col_ref[...] = lax.broadcasted_iota(
        jnp.int32, (_B, _S), 1).astype(jnp.float32)
    iota = iota_ref[...]
    col = col_ref[...]
    nf = jnp.float32(_N)

    def body(t, carry):
        dists, far, cx, cy, cz, outv, lcx, lcy, lcz = carry
        hitf = (col == t.astype(jnp.float32)).astype(jnp.float32)
        outv = outv + hitf * jnp.broadcast_to(far, (_B, _S))
        lcx = lcx + hitf * jnp.broadcast_to(cx, (_B, _S))
        lcy = lcy + hitf * jnp.broadcast_to(cy, (_B, _S))
        lcz = lcz + hitf * jnp.broadcast_to(cz, (_B, _S))
        dx = xs - jnp.broadcast_to(cx, (_B, _N))
        dy = ys - jnp.broadcast_to(cy, (_B, _N))
        dz = zs - jnp.broadcast_to(cz, (_B, _N))
        d = dx * dx + dy * dy + dz * dz
        dists = jnp.minimum(dists, d)
        dv, iv, xv, yv, zv = dists, iota, xs, ys, zs
        w = _N
        while w > 128:
            h = w // 2
            dl, dr = dv[:, :h], dv[:, h:]
            il, ir = iv[:, :h], iv[:, h:]
            take_r = (dr > dl) | ((dr == dl) & (ir < il))
            dv = jnp.where(take_r, dr, dl)
            iv = jnp.where(take_r, ir, il)
            xv = jnp.where(take_r, xv[:, h:], xv[:, :h])
            yv = jnp.where(take_r, yv[:, h:], yv[:, :h])
            zv = jnp.where(take_r, zv[:, h:], zv[:, :h])
            w = h
        m = jnp.max(dv, axis=1, keepdims=True)
        candm = jnp.where(dv == jnp.broadcast_to(m, (_B, 128)), iv, nf)
        far2 = jnp.min(candm, axis=1, keepdims=True)
        pos = (iv == jnp.broadcast_to(far2, (_B, 128))).astype(jnp.float32)
        cx2 = jnp.sum(pos * xv, axis=1, keepdims=True)
        cy2 = jnp.sum(pos * yv, axis=1, keepdims=True)
        cz2 = jnp.sum(pos * zv, axis=1, keepdims=True)
        return dists, far2, cx2, cy2, cz2, outv, lcx, lcy, lcz

    dists0 = jnp.full((_B, _N), 1e10, jnp.float32)
    far0 = jnp.zeros((_B, 1), jnp.float32)
    cx0 = xs[:, 0:1]
    cy0 = ys[:, 0:1]
    cz0 = zs[:, 0:1]
    z = jnp.zeros((_B, _S), jnp.float32)
    out = lax.fori_loop(
        0, _S, body, (dists0, far0, cx0, cy0, cz0, z, z, z, z))
    _, _, _, _, _, outv, lcx, lcy, lcz = out
    out_ref[...] = outv.astype(jnp.int32)
    q_ref[0:_B, :] = lcx
    q_ref[_B:2 * _B, :] = lcy
    q_ref[2 * _B:3 * _B, :] = lcz


def _fps(pts):
    return pl.pallas_call(
        _fps_body,
        out_shape=(
            jax.ShapeDtypeStruct((_B, _S), jnp.int32),
            jax.ShapeDtypeStruct((3 * _B, _S), jnp.float32),
        ),
        scratch_shapes=[
            pltpu.VMEM((_B, _N), jnp.float32),
            pltpu.VMEM((_B, _S), jnp.float32),
        ],
    )(pts)


def _knn_body(q_ref, p_ref, out_ref, dist_ref, iota_ref, kcol_ref):
    px = p_ref[0, 0]
    py = p_ref[1, 0]
    pz = p_ref[2, 0]
    qx = q_ref[0, 0]
    qy = q_ref[1, 0]
    qz = q_ref[2, 0]
    def rb(v):
        return v.astype(jnp.bfloat16).astype(jnp.float32)

    dot = (rb(qx) * rb(px) + rb(qy) * rb(py)) + rb(qz) * rb(pz)
    q2 = (qx * qx + qy * qy) + qz * qz
    p2 = (px * px + py * py) + pz * pz
    dist_ref[...] = (-2.0 * dot + q2) + p2

    iota_ref[...] = lax.broadcasted_iota(
        jnp.int32, (_QB, _N), 1).astype(jnp.float32)
    kcol_ref[...] = lax.broadcasted_iota(jnp.int32, (_QB, _K), 1)
    iota = iota_ref[...]
    kcol = kcol_ref[...]
    big = jnp.float32(1e30)
    nf = jnp.float32(_N)

    def body(k, outv):
        d = dist_ref[...]
        m = jnp.min(d, axis=1, keepdims=True)
        cand = jnp.where(d == jnp.broadcast_to(m, (_QB, _N)), iota, nf)
        idx = jnp.min(cand, axis=1, keepdims=True)
        hiti = (kcol == k).astype(jnp.int32)
        outv = outv + hiti * jnp.broadcast_to(
            idx.astype(jnp.int32), (_QB, _K))
        picked = jnp.broadcast_to(idx, (_QB, _N)) == iota
        dist_ref[...] = jnp.where(picked, big, d)
        return outv

    outv = lax.fori_loop(0, _K, body, jnp.zeros((_QB, _K), jnp.int32))
    out_ref[...] = outv.reshape(1, _QB, _K)


def _knn(q, pts4):
    return pl.pallas_call(
        _knn_body,
        grid=(_B, _S // _QB),
        in_specs=[
            pl.BlockSpec((3, 1, _QB, 1), lambda b, j: (0, b, j, 0)),
            pl.BlockSpec((3, 1, 1, _N), lambda b, j: (0, b, 0, 0)),
        ],
        out_specs=pl.BlockSpec((1, _QB, _K), lambda b, j: (b, j, 0)),
        out_shape=jax.ShapeDtypeStruct((_B, _S, _K), jnp.int32),
        scratch_shapes=[
            pltpu.VMEM((_QB, _N), jnp.float32),
            pltpu.VMEM((_QB, _N), jnp.float32),
            pltpu.VMEM((_QB, _K), jnp.int32),
        ],
    )(q, pts4)


@functools.lru_cache(maxsize=None)
def _make_sc_gather(n_idx, n_tables):
    info = plsc.get_sparse_core_info()
    nc, ns = info.num_cores, info.num_subcores
    nw = nc * ns
    per_w = n_idx // nw
    ch = min(per_w, 128)
    chunks = per_w // ch
    grp = min(2, chunks)
    groups = chunks // grp
    mesh = plsc.VectorSubcoreMesh(core_axis_name="c", subcore_axis_name="s")

    @functools.partial(
        pl.kernel,
        mesh=mesh,
        out_type=tuple(
            jax.ShapeDtypeStruct((n_idx, _C), jnp.float32)
            for _ in range(n_tables)
        ),
        scratch_types=(
            [pltpu.VMEM((grp * ch,), jnp.int32)]
            + [pltpu.VMEM((grp * ch, _C), jnp.float32)
               for _ in range(n_tables)]
            + [pltpu.SemaphoreType.DMA for _ in range(n_tables)]
        ),
    )
    def k(idx_hbm, *rest):
        tables = rest[:n_tables]
        outs = rest[n_tables:2 * n_tables]
        idx_v = rest[2 * n_tables]
        rows = rest[2 * n_tables + 1:2 * n_tables + 1 + n_tables]
        sems = rest[2 * n_tables + 1 + n_tables:]
        wid = lax.axis_index("s") * nc + lax.axis_index("c")
        base = wid * per_w

        def body(g, carry):
            off = base + g * grp * ch
            pltpu.sync_copy(idx_hbm.at[pl.ds(off, grp * ch)], idx_v)
            cps = []
            for j in range(grp):
                sl = pl.ds(j * ch, ch)
                for t, r, s in zip(tables, rows, sems):
                    cps.append(pltpu.async_copy(t.at[idx_v.at[sl]],
                                                r.at[sl], s))
            for cp in cps:
                cp.wait()
            for r, o in zip(rows, outs):
                pltpu.sync_copy(r, o.at[pl.ds(off, grp * ch)])
            return carry

        lax.fori_loop(0, groups, body, 0)

    return k


def kernel(xyz, x):
    pts = jnp.transpose(xyz, (2, 0, 1)).reshape(3 * _B, _N)
    pts4 = pts.reshape(3, _B, 1, _N)
    x2d = x.reshape(_B * _N, _C)
    xyz128 = jnp.pad(xyz, ((0, 0), (0, 0), (0, _C - 3)))
    xyz128 = xyz128.reshape(_B * _N, _C)
    row_off = jnp.arange(_B, dtype=jnp.int32) * _N

    fps_idx, qpts = _fps(pts)
    lc_xyz = qpts.reshape(3, _B, _S).transpose(1, 2, 0)

    gidx_lc = (fps_idx + row_off[:, None]).reshape(-1)
    (lc_x_flat,) = _make_sc_gather(_B * _S, 1)(gidx_lc, x2d)
    lc_x = lc_x_flat.reshape(_B, _S, _C)

    q = qpts.reshape(3, _B, _S, 1)
    knn_idx = _knn(q, pts4)

    gidx_knn = (knn_idx + row_off[:, None, None]).reshape(-1)
    knn_x_flat, knn_xyz_flat = _make_sc_gather(_B * _S * _K, 2)(
        gidx_knn, x2d, xyz128)
    knn_x = knn_x_flat.reshape(_B, _S, _K, _C)
    knn_xyz = knn_xyz_flat[:, :3].reshape(_B, _S, _K, 3)

    return (lc_xyz, lc_x, knn_xyz, knn_x)
